# baseline (speedup 1.0000x reference)
import dataclasses
import functools

import jax
import jax.numpy as jnp
from jax import lax
from jax.experimental import pallas as pl
from jax.experimental.pallas import tpu as pltpu
from jax.experimental.pallas import tpu_sc as plsc

_NC = 2
_NS = 16
_L = 16
_C = 80


def _sc_compiler_params():
    cp = pltpu.CompilerParams()
    if "needs_layout_passes" in pltpu.CompilerParams.__dataclass_fields__:
        cp = dataclasses.replace(cp, needs_layout_passes=False)
    return cp


def _sc_degree(col_i32, ew, n_pad, e_pad):
    nw = _NC * _NS
    per_tile = e_pad // nw
    chunks = per_tile // _C
    sb = 8
    n_sb = chunks // sb
    slice_n = n_pad // _NS
    mesh = plsc.VectorSubcoreMesh(core_axis_name="c", subcore_axis_name="s")

    @functools.partial(
        pl.kernel,
        out_type=jax.ShapeDtypeStruct((_NC * n_pad,), jnp.float32),
        mesh=mesh,
        scratch_types=[
            pltpu.VMEM((n_sb, sb, _C), jnp.int32),
            pltpu.VMEM((n_sb, sb, _C), jnp.float32),
            pltpu.VMEM((slice_n,), jnp.float32),
            pltpu.SemaphoreType.DMA,
            pltpu.VMEM_SHARED((n_pad,), jnp.float32),
        ],
    )
    def deg_kernel(col_hbm, ew_hbm, out_hbm, idx_v, ew_v, zbuf, sem, deg_sh):
        c = lax.axis_index("c")
        s = lax.axis_index("s")
        wid = c * _NS + s

        @pl.loop(0, slice_n // _L)
        def _zero(i):
            zbuf[pl.ds(i * _L, _L)] = jnp.zeros((_L,), jnp.float32)

        pltpu.sync_copy(zbuf, deg_sh.at[pl.ds(s * slice_n, slice_n)])
        pltpu.sync_copy(col_hbm.at[pl.ds(wid * n_sb, n_sb)], idx_v)
        pltpu.sync_copy(ew_hbm.at[pl.ds(wid * n_sb, n_sb)], ew_v)
        plsc.subcore_barrier()

        @pl.loop(0, n_sb)
        def _acc(t):
            for j in range(sb):
                pltpu.async_copy(
                    ew_v.at[t, j], deg_sh.at[idx_v.at[t, j]], sem, add=True)
            for j in range(sb):
                pltpu.make_async_copy(
                    ew_v.at[t, j], deg_sh.at[idx_v.at[t, j]], sem).wait()

        plsc.subcore_barrier()
        pltpu.sync_copy(
            deg_sh.at[pl.ds(s * slice_n, slice_n)],
            out_hbm.at[pl.ds(c * n_pad + s * slice_n, slice_n)],
        )

    return deg_kernel(col_i32, ew)


def _sc_aggregate(row_i32, col_i32, ew, xs, n_pad, e_pad, d):
    nw = _NC * _NS
    per_tile = e_pad // nw
    chunks = per_tile // _C
    slice_n = n_pad // _NS
    mesh = plsc.VectorSubcoreMesh(core_axis_name="c", subcore_axis_name="s")

    sb = 8
    n_sb = chunks // sb
    nb = 4

    s_tot = e_pad // (sb * _C)
    m0 = (s_tot * 7 // 8) // _NS
    m1 = (s_tot - s_tot * 7 // 8) // _NS
    assert m0 * _NS + m1 * _NS == s_tot and m0 % 2 == 0 and m1 % 2 == 0

    @functools.partial(
        pl.kernel,
        out_type=jax.ShapeDtypeStruct((_NC * n_pad, d), jnp.float32),
        mesh=mesh,
        scratch_types=[
            pltpu.VMEM((sb, _C), jnp.int32),
            pltpu.VMEM((sb, _C), jnp.int32),
            pltpu.VMEM((sb, _C), jnp.int32),
            pltpu.VMEM((sb, _C), jnp.int32),
            pltpu.VMEM((sb, _C), jnp.float32),
            pltpu.VMEM((sb, _C), jnp.float32),
            pltpu.VMEM((_C, d), jnp.float32),
            pltpu.VMEM((_C, d), jnp.float32),
            pltpu.VMEM((_C, d), jnp.float32),
            pltpu.VMEM((_C, d), jnp.float32),
            pltpu.SemaphoreType.DMA,
            pltpu.SemaphoreType.DMA,
            pltpu.SemaphoreType.DMA,
            pltpu.SemaphoreType.DMA,
            pltpu.SemaphoreType.DMA,
            pltpu.SemaphoreType.DMA,
            pltpu.SemaphoreType.DMA,
            pltpu.SemaphoreType.DMA,
            pltpu.SemaphoreType.DMA,
            pltpu.SemaphoreType.DMA,
            pltpu.VMEM_SHARED((n_pad, d), jnp.float32),
        ],
        compiler_params=_sc_compiler_params(),
    )
    def agg_kernel(row_hbm, col_hbm, ew_hbm, xs_hbm, out_hbm,
                   ridx0, ridx1, cidx0, cidx1, ew0, ew1,
                   rows0, rows1, rows2, rows3,
                   si0, si1, sg0, sg1, sg2, sg3, ss0, ss1, ss2, ss3, p_sh):
        c = lax.axis_index("c")
        s = lax.axis_index("s")
        msb = jnp.where(c == 0, m0, m1)
        lo = jnp.where(c == 0, s * m0, m0 * _NS + s * m1)
        ridx = (ridx0, ridx1)
        cidx = (cidx0, cidx1)
        ew = (ew0, ew1)
        rows = (rows0, rows1, rows2, rows3)
        si = (si0, si1)
        sg = (sg0, sg1, sg2, sg3)
        ss = (ss0, ss1, ss2, ss3)

        @pl.when(c == 0)
        def _init_xs():
            pltpu.sync_copy(
                xs_hbm.at[pl.ds(s * slice_n, slice_n)],
                p_sh.at[pl.ds(s * slice_n, slice_n)],
            )

        @pl.when(c != 0)
        def _init_zero():
            @pl.loop(0, _C)
            def _zrow(i):
                for dd in range(d // _L):
                    rows0[i, pl.ds(dd * _L, _L)] = jnp.zeros((_L,),
                                                             jnp.float32)

            @pl.loop(0, slice_n // _C)
            def _zcopy(i):
                pltpu.sync_copy(
                    rows0, p_sh.at[pl.ds(s * slice_n + i * _C, _C)])

        def start_idx(u, r):
            su = lo + u
            pltpu.async_copy(row_hbm.at[su], ridx[r], si[r])
            pltpu.async_copy(col_hbm.at[su], cidx[r], si[r])
            pltpu.async_copy(ew_hbm.at[su], ew[r], si[r])

        def wait_idx(u, r):
            su = lo + u
            pltpu.make_async_copy(row_hbm.at[su], ridx[r], si[r]).wait()
            pltpu.make_async_copy(col_hbm.at[su], cidx[r], si[r]).wait()
            pltpu.make_async_copy(ew_hbm.at[su], ew[r], si[r]).wait()

        def start_gather(jl, r, b):
            pltpu.async_copy(xs_hbm.at[ridx[r].at[jl]], rows[b], sg[b])

        def wait_gather(jl, r, b):
            pltpu.make_async_copy(xs_hbm.at[ridx[r].at[jl]], rows[b],
                                  sg[b]).wait()

        def start_scatter(jl, r, b):
            pltpu.async_copy(rows[b], p_sh.at[cidx[r].at[jl]], ss[b],
                             add=True)

        def wait_scatter(jl, r, b):
            pltpu.make_async_copy(rows[b], p_sh.at[cidx[r].at[jl]],
                                  ss[b]).wait()

        def scale(jl, r, b):
            rb = rows[b]
            ewr = ew[r]

            @pl.loop(0, _C, step=4)
            def _scale(j):
                jlv = jnp.full((_L,), jl, jnp.int32)
                zv = jnp.full((_L,), 0, jnp.int32)
                wv = [plsc.load_gather(ewr, [jlv, zv + (j + u)])
                      for u in range(4)]
                for dd in range(d // _L):
                    for u in range(4):
                        sl = (j + u, pl.ds(dd * _L, _L))
                        rb[sl] = rb[sl] * wv[u]

        start_idx(0, 0)
        wait_idx(0, 0)
        plsc.subcore_barrier()
        start_gather(0, 0, 0)
        start_gather(1, 0, 1)

        def _pair(v, carry):
            for r in range(2):
                u = v * 2 + r
                for j in range(sb):
                    b = j % nb
                    bn = (j + 2) % nb

                    if j >= 2:
                        wait_scatter(j - 2, r, bn)
                    else:
                        @pl.when(u > 0)
                        def _(): wait_scatter(sb - 2 + j, 1 - r, bn)

                    if j == 2:
                        @pl.when(u + 1 < msb)
                        def _(): start_idx(u + 1, 1 - r)

                    if j < sb - 2:
                        start_gather(j + 2, r, bn)
                    else:
                        if j == sb - 2:
                            @pl.when(u + 1 < msb)
                            def _(): wait_idx(u + 1, 1 - r)
                        @pl.when(u + 1 < msb)
                        def _(): start_gather(j + 2 - sb, 1 - r, bn)

                    wait_gather(j, r, b)
                    scale(j, r, b)
                    start_scatter(j, r, b)
            return carry

        lax.fori_loop(0, msb // 2, _pair, 0)
        wait_scatter(sb - 2, 1, (sb - 2) % nb)
        wait_scatter(sb - 1, 1, (sb - 1) % nb)
        plsc.subcore_barrier()
        pltpu.sync_copy(
            p_sh.at[pl.ds(s * slice_n, slice_n)],
            out_hbm.at[pl.ds(c * n_pad + s * slice_n, slice_n)],
        )

    return agg_kernel(row_i32, col_i32, ew, xs)


def _tc_scale(dp0, dp1, x_pad):
    n_pad, d = x_pad.shape
    blk = 2048

    def body(dp0_ref, dp1_ref, x_ref, dis_ref, xs_ref):
        deg = 1.0 + dp0_ref[...] + dp1_ref[...]
        dis = lax.rsqrt(deg)
        dis_ref[...] = dis
        xs_ref[...] = x_ref[...] * dis

    return pl.pallas_call(
        body,
        grid=(n_pad // blk,),
        in_specs=[
            pl.BlockSpec((blk, 1), lambda i: (i, 0)),
            pl.BlockSpec((blk, 1), lambda i: (i, 0)),
            pl.BlockSpec((blk, d), lambda i: (i, 0)),
        ],
        out_specs=[
            pl.BlockSpec((blk, 1), lambda i: (i, 0)),
            pl.BlockSpec((blk, d), lambda i: (i, 0)),
        ],
        out_shape=[
            jax.ShapeDtypeStruct((n_pad, 1), jnp.float32),
            jax.ShapeDtypeStruct((n_pad, d), jnp.float32),
        ],
    )(dp0, dp1, x_pad)


def _tc_head(p0, p1, dis, w1, b1, w2, b2):
    n_pad, d = p0.shape
    h_dim = w1.shape[0]
    o_dim = w2.shape[0]
    blk = 1024

    def body(p0_ref, p1_ref, dis_ref, w1_ref, b1_ref, w2_ref, b2_ref,
             logits_ref, h_ref):
        m = dis_ref[...] * (p0_ref[...] + p1_ref[...])
        h = lax.dot_general(m, w1_ref[...], (((1,), (1,)), ((), ())),
                            preferred_element_type=jnp.float32)
        h = jnp.maximum(h + b1_ref[...], 0.0)
        h_ref[...] = h
        logits_ref[...] = lax.dot_general(
            h, w2_ref[...], (((1,), (1,)), ((), ())),
            preferred_element_type=jnp.float32) + b2_ref[...]

    return pl.pallas_call(
        body,
        grid=(n_pad // blk,),
        in_specs=[
            pl.BlockSpec((blk, d), lambda i: (i, 0)),
            pl.BlockSpec((blk, d), lambda i: (i, 0)),
            pl.BlockSpec((blk, 1), lambda i: (i, 0)),
            pl.BlockSpec((h_dim, d), lambda i: (0, 0)),
            pl.BlockSpec((1, h_dim), lambda i: (0, 0)),
            pl.BlockSpec((o_dim, h_dim), lambda i: (0, 0)),
            pl.BlockSpec((1, o_dim), lambda i: (0, 0)),
        ],
        out_specs=[
            pl.BlockSpec((blk, o_dim), lambda i: (i, 0)),
            pl.BlockSpec((blk, h_dim), lambda i: (i, 0)),
        ],
        out_shape=[
            jax.ShapeDtypeStruct((n_pad, o_dim), jnp.float32),
            jax.ShapeDtypeStruct((n_pad, h_dim), jnp.float32),
        ],
    )(p0, p1, dis, w1, b1, w2, b2)


def kernel(x, edge_index, edge_weight, W1, b1, W2, b2):
    n, d = x.shape
    e = edge_weight.shape[0]
    nw = _NC * _NS

    blk = 2048
    n_pad = -(-n // blk) * blk
    per_tile = -(-e // (nw * 16 * _C)) * 16 * _C
    e_pad = per_tile * nw

    row = edge_index[0].astype(jnp.int32)
    col = edge_index[1].astype(jnp.int32)
    row_p = jnp.pad(row, (0, e_pad - e)).reshape(-1, 8, _C)
    col_p = jnp.pad(col, (0, e_pad - e)).reshape(-1, 8, _C)
    ew_p = jnp.pad(edge_weight.astype(jnp.float32), (0, e_pad - e)).reshape(-1, 8, _C)
    x_p = jnp.pad(x, ((0, n_pad - n), (0, 0)))

    deg_part = _sc_degree(col_p, ew_p, n_pad, e_pad)
    dp0 = deg_part[:n_pad].reshape(n_pad, 1)
    dp1 = deg_part[n_pad:].reshape(n_pad, 1)
    dis, xs = _tc_scale(dp0, dp1, x_p)

    p = _sc_aggregate(row_p, col_p, ew_p, xs, n_pad, e_pad, d)
    logits_pad, h_pad = _tc_head(
        p[:n_pad], p[n_pad:], dis,
        W1, b1.reshape(1, -1), W2, b2.reshape(1, -1))
    return (logits_pad[:n], h_pad[:n])


# speedup vs baseline: 1.0382x; 1.0382x over previous
import dataclasses
import functools

import jax
import jax.numpy as jnp
from jax import lax
from jax.experimental import pallas as pl
from jax.experimental.pallas import tpu as pltpu
from jax.experimental.pallas import tpu_sc as plsc

_NC = 2
_NS = 16
_L = 16
_C = 80


def _sc_compiler_params():
    cp = pltpu.CompilerParams()
    if "needs_layout_passes" in pltpu.CompilerParams.__dataclass_fields__:
        cp = dataclasses.replace(cp, needs_layout_passes=False)
    return cp


def _sc_degree(col_i32, ew, n_pad, e_pad):
    nw = _NC * _NS
    per_tile = e_pad // nw
    chunks = per_tile // _C
    sb = 8
    n_sb = chunks // sb
    slice_n = n_pad // _NS
    mesh = plsc.VectorSubcoreMesh(core_axis_name="c", subcore_axis_name="s")

    @functools.partial(
        pl.kernel,
        out_type=jax.ShapeDtypeStruct((_NC * n_pad,), jnp.float32),
        mesh=mesh,
        scratch_types=[
            pltpu.VMEM((n_sb, sb, _C), jnp.int32),
            pltpu.VMEM((n_sb, sb, _C), jnp.float32),
            pltpu.VMEM((slice_n,), jnp.float32),
            pltpu.SemaphoreType.DMA,
            pltpu.VMEM_SHARED((n_pad,), jnp.float32),
        ],
    )
    def deg_kernel(col_hbm, ew_hbm, out_hbm, idx_v, ew_v, zbuf, sem, deg_sh):
        c = lax.axis_index("c")
        s = lax.axis_index("s")
        wid = c * _NS + s

        @pl.loop(0, slice_n // _L)
        def _zero(i):
            zbuf[pl.ds(i * _L, _L)] = jnp.zeros((_L,), jnp.float32)

        pltpu.sync_copy(zbuf, deg_sh.at[pl.ds(s * slice_n, slice_n)])
        pltpu.sync_copy(col_hbm.at[pl.ds(wid * n_sb, n_sb)], idx_v)
        pltpu.sync_copy(ew_hbm.at[pl.ds(wid * n_sb, n_sb)], ew_v)
        plsc.subcore_barrier()

        @pl.loop(0, n_sb)
        def _acc(t):
            for j in range(sb):
                pltpu.async_copy(
                    ew_v.at[t, j], deg_sh.at[idx_v.at[t, j]], sem, add=True)
            for j in range(sb):
                pltpu.make_async_copy(
                    ew_v.at[t, j], deg_sh.at[idx_v.at[t, j]], sem).wait()

        plsc.subcore_barrier()
        pltpu.sync_copy(
            deg_sh.at[pl.ds(s * slice_n, slice_n)],
            out_hbm.at[pl.ds(c * n_pad + s * slice_n, slice_n)],
        )

    return deg_kernel(col_i32, ew)


def _sc_aggregate(row_i32, col_i32, ew, xs, n_pad, e_pad, d):
    nw = _NC * _NS
    per_tile = e_pad // nw
    chunks = per_tile // _C
    slice_n = n_pad // _NS
    mesh = plsc.VectorSubcoreMesh(core_axis_name="c", subcore_axis_name="s")

    sb = 8
    n_sb = chunks // sb
    nb = 4

    s_tot = e_pad // (sb * _C)
    m0 = (s_tot * 7 // 8) // _NS
    m1 = (s_tot - s_tot * 7 // 8) // _NS
    assert m0 * _NS + m1 * _NS == s_tot and m0 % 2 == 0 and m1 % 2 == 0

    @functools.partial(
        pl.kernel,
        out_type=jax.ShapeDtypeStruct((_NC * n_pad, d), jnp.float32),
        mesh=mesh,
        scratch_types=[
            pltpu.VMEM((sb, _C), jnp.int32),
            pltpu.VMEM((sb, _C), jnp.int32),
            pltpu.VMEM((sb, _C), jnp.int32),
            pltpu.VMEM((sb, _C), jnp.int32),
            pltpu.VMEM((sb, _C), jnp.float32),
            pltpu.VMEM((sb, _C), jnp.float32),
            pltpu.VMEM((_C, d), jnp.float32),
            pltpu.VMEM((_C, d), jnp.float32),
            pltpu.VMEM((_C, d), jnp.float32),
            pltpu.VMEM((_C, d), jnp.float32),
            pltpu.SemaphoreType.DMA,
            pltpu.SemaphoreType.DMA,
            pltpu.SemaphoreType.DMA,
            pltpu.SemaphoreType.DMA,
            pltpu.SemaphoreType.DMA,
            pltpu.SemaphoreType.DMA,
            pltpu.SemaphoreType.DMA,
            pltpu.SemaphoreType.DMA,
            pltpu.SemaphoreType.DMA,
            pltpu.SemaphoreType.DMA,
            pltpu.VMEM_SHARED((n_pad, d), jnp.float32),
        ],
        compiler_params=_sc_compiler_params(),
    )
    def agg_kernel(row_hbm, col_hbm, ew_hbm, xs_hbm, out_hbm,
                   ridx0, ridx1, cidx0, cidx1, ew0, ew1,
                   rows0, rows1, rows2, rows3,
                   si0, si1, sg0, sg1, sg2, sg3, ss0, ss1, ss2, ss3, p_sh):
        c = lax.axis_index("c")
        s = lax.axis_index("s")
        msb = jnp.where(c == 0, m0, m1)
        lo = jnp.where(c == 0, s * m0, m0 * _NS + s * m1)
        ridx = (ridx0, ridx1)
        cidx = (cidx0, cidx1)
        ew = (ew0, ew1)
        rows = (rows0, rows1, rows2, rows3)
        si = (si0, si1)
        sg = (sg0, sg1, sg2, sg3)
        ss = (ss0, ss1, ss2, ss3)

        pltpu.sync_copy(
            xs_hbm.at[pl.ds(s * slice_n, slice_n)],
            p_sh.at[pl.ds(s * slice_n, slice_n)],
        )

        def start_idx(u, r):
            su = lo + u
            pltpu.async_copy(row_hbm.at[su], ridx[r], si[r])
            pltpu.async_copy(col_hbm.at[su], cidx[r], si[r])
            pltpu.async_copy(ew_hbm.at[su], ew[r], si[r])

        def wait_idx(u, r):
            su = lo + u
            pltpu.make_async_copy(row_hbm.at[su], ridx[r], si[r]).wait()
            pltpu.make_async_copy(col_hbm.at[su], cidx[r], si[r]).wait()
            pltpu.make_async_copy(ew_hbm.at[su], ew[r], si[r]).wait()

        def start_gather(jl, r, b):
            pltpu.async_copy(xs_hbm.at[ridx[r].at[jl]], rows[b], sg[b])

        def wait_gather(jl, r, b):
            pltpu.make_async_copy(xs_hbm.at[ridx[r].at[jl]], rows[b],
                                  sg[b]).wait()

        def start_scatter(jl, r, b):
            pltpu.async_copy(rows[b], p_sh.at[cidx[r].at[jl]], ss[b],
                             add=True)

        def wait_scatter(jl, r, b):
            pltpu.make_async_copy(rows[b], p_sh.at[cidx[r].at[jl]],
                                  ss[b]).wait()

        def scale(jl, r, b):
            rb = rows[b]
            ewr = ew[r]

            @pl.loop(0, _C, step=4)
            def _scale(j):
                jlv = jnp.full((_L,), jl, jnp.int32)
                zv = jnp.full((_L,), 0, jnp.int32)
                wv = [plsc.load_gather(ewr, [jlv, zv + (j + u)])
                      for u in range(4)]
                for dd in range(d // _L):
                    for u in range(4):
                        sl = (j + u, pl.ds(dd * _L, _L))
                        rb[sl] = rb[sl] * wv[u]

        start_idx(0, 0)
        wait_idx(0, 0)
        plsc.subcore_barrier()
        start_gather(0, 0, 0)
        start_gather(1, 0, 1)

        def _pair(v, carry):
            for r in range(2):
                u = v * 2 + r
                for j in range(sb):
                    b = j % nb
                    bn = (j + 2) % nb

                    if j >= 2:
                        wait_scatter(j - 2, r, bn)
                    else:
                        @pl.when(u > 0)
                        def _(): wait_scatter(sb - 2 + j, 1 - r, bn)

                    if j == 2:
                        @pl.when(u + 1 < msb)
                        def _(): start_idx(u + 1, 1 - r)

                    if j < sb - 2:
                        start_gather(j + 2, r, bn)
                    else:
                        if j == sb - 2:
                            @pl.when(u + 1 < msb)
                            def _(): wait_idx(u + 1, 1 - r)
                        @pl.when(u + 1 < msb)
                        def _(): start_gather(j + 2 - sb, 1 - r, bn)

                    wait_gather(j, r, b)
                    scale(j, r, b)
                    start_scatter(j, r, b)
            return carry

        lax.fori_loop(0, msb // 2, _pair, 0)
        wait_scatter(sb - 2, 1, (sb - 2) % nb)
        wait_scatter(sb - 1, 1, (sb - 1) % nb)
        plsc.subcore_barrier()
        pltpu.sync_copy(
            p_sh.at[pl.ds(s * slice_n, slice_n)],
            out_hbm.at[pl.ds(c * n_pad + s * slice_n, slice_n)],
        )

    return agg_kernel(row_i32, col_i32, ew, xs)


def _tc_scale(dp0, dp1, x_pad):
    n_pad, d = x_pad.shape
    blk = 2048

    def body(dp0_ref, dp1_ref, x_ref, dis_ref, xs_ref):
        deg = 1.0 + dp0_ref[...] + dp1_ref[...]
        dis = lax.rsqrt(deg)
        dis_ref[...] = dis
        xs_ref[...] = x_ref[...] * dis

    return pl.pallas_call(
        body,
        grid=(n_pad // blk,),
        in_specs=[
            pl.BlockSpec((blk, 1), lambda i: (i, 0)),
            pl.BlockSpec((blk, 1), lambda i: (i, 0)),
            pl.BlockSpec((blk, d), lambda i: (i, 0)),
        ],
        out_specs=[
            pl.BlockSpec((blk, 1), lambda i: (i, 0)),
            pl.BlockSpec((blk, d), lambda i: (i, 0)),
        ],
        out_shape=[
            jax.ShapeDtypeStruct((n_pad, 1), jnp.float32),
            jax.ShapeDtypeStruct((n_pad, d), jnp.float32),
        ],
    )(dp0, dp1, x_pad)


def _tc_head(p0, p1, xs, dis, w1, b1, w2, b2):
    n_pad, d = xs.shape
    h_dim = w1.shape[0]
    o_dim = w2.shape[0]
    blk = 1024

    def body(p0_ref, p1_ref, xs_ref, dis_ref, w1_ref, b1_ref, w2_ref, b2_ref,
             logits_ref, h_ref):
        m = dis_ref[...] * (p0_ref[...] + p1_ref[...] - xs_ref[...])
        h = lax.dot_general(m, w1_ref[...], (((1,), (1,)), ((), ())),
                            preferred_element_type=jnp.float32)
        h = jnp.maximum(h + b1_ref[...], 0.0)
        h_ref[...] = h
        logits_ref[...] = lax.dot_general(
            h, w2_ref[...], (((1,), (1,)), ((), ())),
            preferred_element_type=jnp.float32) + b2_ref[...]

    return pl.pallas_call(
        body,
        grid=(n_pad // blk,),
        in_specs=[
            pl.BlockSpec((blk, d), lambda i: (i, 0)),
            pl.BlockSpec((blk, d), lambda i: (i, 0)),
            pl.BlockSpec((blk, d), lambda i: (i, 0)),
            pl.BlockSpec((blk, 1), lambda i: (i, 0)),
            pl.BlockSpec((h_dim, d), lambda i: (0, 0)),
            pl.BlockSpec((1, h_dim), lambda i: (0, 0)),
            pl.BlockSpec((o_dim, h_dim), lambda i: (0, 0)),
            pl.BlockSpec((1, o_dim), lambda i: (0, 0)),
        ],
        out_specs=[
            pl.BlockSpec((blk, o_dim), lambda i: (i, 0)),
            pl.BlockSpec((blk, h_dim), lambda i: (i, 0)),
        ],
        out_shape=[
            jax.ShapeDtypeStruct((n_pad, o_dim), jnp.float32),
            jax.ShapeDtypeStruct((n_pad, h_dim), jnp.float32),
        ],
    )(p0, p1, xs, dis, w1, b1, w2, b2)


def kernel(x, edge_index, edge_weight, W1, b1, W2, b2):
    n, d = x.shape
    e = edge_weight.shape[0]
    nw = _NC * _NS

    blk = 2048
    n_pad = -(-n // blk) * blk
    per_tile = -(-e // (nw * 16 * _C)) * 16 * _C
    e_pad = per_tile * nw

    row = edge_index[0].astype(jnp.int32)
    col = edge_index[1].astype(jnp.int32)
    row_p = jnp.pad(row, (0, e_pad - e)).reshape(-1, 8, _C)
    col_p = jnp.pad(col, (0, e_pad - e)).reshape(-1, 8, _C)
    ew_p = jnp.pad(edge_weight.astype(jnp.float32), (0, e_pad - e)).reshape(-1, 8, _C)
    x_p = jnp.pad(x, ((0, n_pad - n), (0, 0)))

    deg_part = _sc_degree(col_p, ew_p, n_pad, e_pad)
    dp0 = deg_part[:n_pad].reshape(n_pad, 1)
    dp1 = deg_part[n_pad:].reshape(n_pad, 1)
    dis, xs = _tc_scale(dp0, dp1, x_p)

    p = _sc_aggregate(row_p, col_p, ew_p, xs, n_pad, e_pad, d)
    logits_pad, h_pad = _tc_head(
        p[:n_pad], p[n_pad:], xs, dis,
        W1, b1.reshape(1, -1), W2, b2.reshape(1, -1))
    return (logits_pad[:n], h_pad[:n])
